# baseline (speedup 1.0000x reference)
import functools
import jax
import jax.numpy as jnp
from jax import lax
from jax.experimental import pallas as pl
from jax.experimental.pallas import tpu as pltpu
from jax.experimental.pallas import tpu_sc as plsc

B = 16384
D = 32
L = 16
NC = 2
NS = 16
NW = NC * NS
BPW = B // NW
Q = 128
NQ = BPW // Q
QB = Q // L
V = 1000000
LINES = V // 4
CL = 64
LPT = 15616
NCHUNK = LPT // CL
EXTRA = LINES - 16 * LPT
EXTRA_CHUNKS = (64, 64, 16)

_GD = lax.GatherDimensionNumbers(
    offset_dims=(), collapsed_slice_dims=(0,), start_index_map=(0,))


def _perm(x, perm):
    return lax.gather(x, perm[:, None], _GD, (1,),
                      mode=lax.GatherScatterMode.PROMISE_IN_BOUNDS)


def _fmt_body(utab_hbm, itab_hbm, ulin_hbm, ilin_hbm,
              rb0, rb1, lb0, lb1,
              rs0, rs1, ws0, ws1):
    cid = lax.axis_index("c")
    sid = lax.axis_index("s")
    wid = sid * NC + cid

    rbufs = [rb0, rb1]
    lbufs = [lb0, lb1]
    rsems = [rs0, rs1]
    wsems = [ws0, ws1]

    def convert(tab_hbm, lin_hbm, tile16):
        lbase = tile16 * LPT

        def read(c, s):
            pltpu.async_copy(tab_hbm.at[pl.ds((lbase + c * CL) * 4, CL * 4), :],
                             rbufs[s], rsems[s])

        def repack(s):
            for l in range(CL):
                for g in range(4):
                    for h in range(2):
                        lbufs[s][l, pl.ds(g * 32 + h * L, L)] = (
                            rbufs[s][l * 4 + g, pl.ds(h * L, L)])

        read(0, 0)
        read(1, 1)

        def body(c2, carry):
            for s in range(2):
                c = c2 * 2 + s
                pltpu.make_async_copy(tab_hbm.at[pl.ds(0, CL * 4), :],
                                      rbufs[s], rsems[s]).wait()
                @pl.when(c >= 2)
                def _(s=s):
                    pltpu.make_async_copy(lbufs[s],
                                          lin_hbm.at[pl.ds(0, CL), :],
                                          wsems[s]).wait()
                repack(s)
                pltpu.async_copy(lbufs[s],
                                 lin_hbm.at[pl.ds(lbase + c * CL, CL), :],
                                 wsems[s])

                @pl.when(c + 2 < NCHUNK)
                def _(s=s, c=c):
                    read(c + 2, s)
            return carry

        lax.fori_loop(0, NCHUNK // 2, body, 0)

        for s in range(2):
            pltpu.make_async_copy(lbufs[s], lin_hbm.at[pl.ds(0, CL), :],
                                  wsems[s]).wait()

        @pl.when(tile16 == 0)
        def _():
            off = 16 * LPT
            for tcl in EXTRA_CHUNKS:
                pltpu.sync_copy(tab_hbm.at[pl.ds(off * 4, tcl * 4), :],
                                rbufs[0].at[pl.ds(0, tcl * 4), :])
                for l in range(tcl):
                    for g in range(4):
                        for h in range(2):
                            lbufs[0][l, pl.ds(g * 32 + h * L, L)] = (
                                rbufs[0][l * 4 + g, pl.ds(h * L, L)])
                pltpu.sync_copy(lbufs[0].at[pl.ds(0, tcl), :],
                                lin_hbm.at[pl.ds(off, tcl), :])
                off += tcl

    @pl.when(wid < 16)
    def _():
        convert(utab_hbm, ulin_hbm, wid)

    @pl.when(wid >= 16)
    def _():
        convert(itab_hbm, ilin_hbm, wid - 16)


_fmt = functools.partial(
    pl.kernel,
    out_type=(jax.ShapeDtypeStruct((LINES, 128), jnp.float32),
              jax.ShapeDtypeStruct((LINES, 128), jnp.float32)),
    mesh=plsc.VectorSubcoreMesh(core_axis_name="c", subcore_axis_name="s"),
    scratch_types=[
        pltpu.VMEM((CL * 4, D), jnp.float32),
        pltpu.VMEM((CL * 4, D), jnp.float32),
        pltpu.VMEM((CL, 128), jnp.float32),
        pltpu.VMEM((CL, 128), jnp.float32),
        pltpu.SemaphoreType.DMA,
        pltpu.SemaphoreType.DMA,
        pltpu.SemaphoreType.DMA,
        pltpu.SemaphoreType.DMA,
    ],
)(_fmt_body)


def _nfm_body(uidx_hbm, iidx_hbm, ulin_hbm, ilin_hbm, w_hbm, out_hbm,
              uidx_v, iidx_v, ulidx_v, ilidx_v, ua, ub, ia, ib, w_v, out_v,
              usemA, usemB, isemA, isemB):
    cid = lax.axis_index("c")
    sid = lax.axis_index("s")
    wid = sid * NC + cid
    base = wid * BPW

    pltpu.sync_copy(uidx_hbm.at[pl.ds(base, BPW)], uidx_v)
    pltpu.sync_copy(iidx_hbm.at[pl.ds(base, BPW)], iidx_v)

    for j in range(BPW // L):
        ulidx_v[pl.ds(j * L, L)] = uidx_v[pl.ds(j * L, L)] >> 2
        ilidx_v[pl.ds(j * L, L)] = iidx_v[pl.ds(j * L, L)] >> 2

    ubufs = [ua, ub]
    ibufs = [ia, ib]
    usems = [usemA, usemB]
    isems = [isemA, isemB]

    def fire(q):
        pltpu.async_copy(ulin_hbm.at[ulidx_v.at[pl.ds(q * Q, Q)]],
                         ubufs[q % 2], usems[q % 2])
        pltpu.async_copy(ilin_hbm.at[ilidx_v.at[pl.ds(q * Q, Q)]],
                         ibufs[q % 2], isems[q % 2])

    def drain(q):
        pltpu.make_async_copy(ulin_hbm.at[pl.ds(0, Q), :],
                              ubufs[q % 2], usems[q % 2]).wait()
        pltpu.make_async_copy(ilin_hbm.at[pl.ds(0, Q), :],
                              ibufs[q % 2], isems[q % 2]).wait()

    fire(0)
    fire(1)

    pltpu.sync_copy(w_hbm, w_v)

    def wrow(base_row, j, half):
        return w_v[base_row + j // 4, pl.ds((j % 4) * 32 + half * L, L)]

    s0 = jnp.zeros((L,), jnp.float32)
    s1 = jnp.zeros((L,), jnp.float32)
    for j in range(D):
        s0 = s0 + wrow(8, j, 0)
        s1 = s1 + wrow(8, j, 1)

    v0 = jnp.zeros((L,), jnp.float32)
    v1 = jnp.zeros((L,), jnp.float32)
    for j in range(D):
        sj = s0[j] if j < L else s1[j - L]
        v0 = v0 + sj * wrow(0, j, 0)
        v1 = v1 + sj * wrow(0, j, 1)

    b00 = w_v[16, pl.ds(0, L)]
    b01 = w_v[16, pl.ds(L, L)]
    b10 = w_v[16, pl.ds(2 * L, L)]
    b11 = w_v[16, pl.ds(3 * L, L)]
    cvec = b00 * s0 + b01 * s1 + b10 + b11
    cc = cvec[0]
    for l in range(1, L):
        cc = cc + cvec[l]

    lane = lax.iota(jnp.int32, L)
    perms = [lane ^ m for m in (1, 2, 4, 8)]
    masks = [lane == j for j in range(L)]
    three = jnp.full((L,), 3, jnp.int32)

    for q in range(NQ):
        drain(q)
        ubuf, ibuf = ubufs[q % 2], ibufs[q % 2]

        def body(b, carry):
            u16 = uidx_v[pl.ds(q * Q + b * L, L)]
            i16 = iidx_v[pl.ds(q * Q + b * L, L)]
            usub = (u16 & three) << 5
            isub = (i16 & three) << 5
            tvec = jnp.zeros((L,), jnp.float32)
            for j in range(L):
                r = b * L + j
                us = usub[j]
                isx = isub[j]
                u0 = ubuf[r, pl.ds(us, L)]
                u1 = ubuf[r, pl.ds(us + L, L)]
                i0 = ibuf[r, pl.ds(isx, L)]
                i1 = ibuf[r, pl.ds(isx + L, L)]
                y = u0 * i0 * v0 + u1 * i1 * v1
                for p in perms:
                    y = y + _perm(y, p)
                tvec = jnp.where(masks[j], y, tvec)
            pr = 1.0 / (1.0 + jnp.exp(-(tvec + cc)))
            out_v[pl.ds(q * Q + b * L, L)] = pr
            return carry

        lax.fori_loop(0, QB, body, 0)
        if q + 2 < NQ:
            fire(q + 2)

    pltpu.sync_copy(out_v, out_hbm.at[pl.ds(base, BPW)])


_nfm = functools.partial(
    pl.kernel,
    out_type=jax.ShapeDtypeStruct((B,), jnp.float32),
    mesh=plsc.VectorSubcoreMesh(core_axis_name="c", subcore_axis_name="s"),
    scratch_types=[
        pltpu.VMEM((BPW,), jnp.int32),
        pltpu.VMEM((BPW,), jnp.int32),
        pltpu.VMEM((BPW,), jnp.int32),
        pltpu.VMEM((BPW,), jnp.int32),
        pltpu.VMEM((Q, 128), jnp.float32),
        pltpu.VMEM((Q, 128), jnp.float32),
        pltpu.VMEM((Q, 128), jnp.float32),
        pltpu.VMEM((Q, 128), jnp.float32),
        pltpu.VMEM((24, 128), jnp.float32),
        pltpu.VMEM((BPW,), jnp.float32),
        pltpu.SemaphoreType.DMA,
        pltpu.SemaphoreType.DMA,
        pltpu.SemaphoreType.DMA,
        pltpu.SemaphoreType.DMA,
    ],
)(_nfm_body)


@jax.jit
def kernel(user_tensor, item_tensor, user_table, item_table, W0, b0, W1, b1):
    w_pack = jnp.concatenate([
        W0.reshape(8, 128),
        W1.reshape(8, 128),
        jnp.concatenate([b0, b1, jnp.zeros((64,), jnp.float32)]).reshape(1, 128),
        jnp.zeros((7, 128), jnp.float32),
    ], axis=0)
    ulin, ilin = _fmt(user_table, item_table)
    return _nfm(user_tensor.astype(jnp.int32), item_tensor.astype(jnp.int32),
                ulin, ilin, w_pack)
